# baseline (speedup 1.0000x reference)
import dataclasses
import functools

import jax
import jax.numpy as jnp
from jax import lax
from jax.experimental import pallas as pl
from jax.experimental.pallas import tpu as pltpu
from jax.experimental.pallas import tpu_sc as plsc

_NC = 2
_NS = 16
_NW = _NC * _NS
_D = 64
_L = 16
_CHUNK = 128
_TBL = 32768
_THB = _TBL // 2
_EPAD = 144
_RPAD = 80


def _tc_transpose_body(in_ref, out_ref):
    h = _TBL // 2
    a = in_ref[:, 0:h][...].T
    b = in_ref[:, h:_TBL][...].T
    out_ref[...] = jnp.concatenate([a, b], axis=1)


def _tc_transpose(ent_t):
    d, n = ent_t.shape
    n_blocks = (n + _TBL - 1) // _TBL
    return pl.pallas_call(
        _tc_transpose_body,
        grid=(n_blocks,),
        in_specs=[pl.BlockSpec((d, _TBL), lambda i: (0, i))],
        out_specs=pl.BlockSpec((_TBL // 2, 2 * d), lambda i: (i, 0)),
        out_shape=jax.ShapeDtypeStruct((n_blocks * (_TBL // 2), 2 * d),
                                       jnp.float32),
    )(ent_t)


def _sc_score(B):
    b_per_w = B // _NW
    n_chunks = b_per_w // _CHUNK
    n_groups = _CHUNK // _L
    mesh = plsc.VectorSubcoreMesh(core_axis_name="c", subcore_axis_name="s")

    cp = pltpu.CompilerParams(use_tc_tiling_on_sc=False)
    if "needs_layout_passes" in pltpu.CompilerParams.__dataclass_fields__:
        cp = dataclasses.replace(cp, needs_layout_passes=False)

    @functools.partial(
        pl.kernel,
        mesh=mesh,
        compiler_params=cp,
        out_type=jax.ShapeDtypeStruct((B,), jnp.float32),
        scratch_types=[
            pltpu.VMEM((b_per_w,), jnp.int32),
            pltpu.VMEM((b_per_w,), jnp.int32),
            pltpu.VMEM((b_per_w,), jnp.int32),
            pltpu.VMEM((b_per_w, _D), jnp.float32),
            pltpu.VMEM((b_per_w, _D), jnp.float32),
            pltpu.VMEM((b_per_w, _D), jnp.float32),
            pltpu.VMEM((b_per_w,), jnp.float32),
            pltpu.SemaphoreType.DMA,
        ],
    )
    def score_kernel(ent_hbm, rel_hbm, hidx_hbm, tidx_hbm, ridx_hbm, out_hbm,
                     hi_v, ti_v, ri_v,
                     hrow_v, trow_v, rrow_v, out_v, sem):
        wid = lax.axis_index("s") * _NC + lax.axis_index("c")
        base = wid * b_per_w
        src = pl.ds(base, b_per_w)
        pltpu.sync_copy(hidx_hbm.at[src], hi_v)
        pltpu.sync_copy(tidx_hbm.at[src], ti_v)
        pltpu.sync_copy(ridx_hbm.at[src], ri_v)
        def remap(i):
            b = lax.shift_right_logical(i, 15)
            l = i - b * _TBL
            half = jnp.where(l >= _THB, 1, 0).astype(jnp.int32)
            lm = l - half * _THB
            return lax.shift_left(b * _THB + lm, 1) + half

        for s in range(b_per_w // _L):
            sl = pl.ds(s * _L, _L)
            hi_v[sl] = remap(hi_v[sl])
            ti_v[sl] = remap(ti_v[sl])

        iota = lax.iota(jnp.int32, _L)

        copies = []
        for c in range(n_chunks):
            csl = pl.ds(c * _CHUNK, _CHUNK)
            dst = pl.ds(c * _CHUNK, _CHUNK)
            copies.append(pltpu.async_copy(
                ent_hbm.at[hi_v.at[csl]], hrow_v.at[dst], sem))
            copies.append(pltpu.async_copy(
                ent_hbm.at[ti_v.at[csl]], trow_v.at[dst], sem))
            copies.append(pltpu.async_copy(
                rel_hbm.at[ri_v.at[csl]], rrow_v.at[dst], sem))
        for cpy in copies:
            cpy.wait()

        @pl.loop(0, b_per_w // _L)
        def _(g):
            lanes = iota + g * _L
            rsl = pl.ds(g * _L, _L)
            out_v[rsl] = jnp.zeros((_L,), jnp.float32)

            @pl.loop(0, _D, step=8)
            def _(d0):
                acc = jnp.zeros((_L,), jnp.float32)
                for dd in range(8):
                    dv = jnp.zeros((_L,), jnp.int32) + (d0 + dd)
                    hv = plsc.load_gather(hrow_v, [lanes, dv])
                    tv = plsc.load_gather(trow_v, [lanes, dv])
                    rv = plsc.load_gather(rrow_v, [lanes, dv])
                    acc = acc + jnp.abs(hv + rv - tv)
                plsc.addupdate(out_v.at[rsl], acc)

        pltpu.sync_copy(out_v, out_hbm.at[pl.ds(base, b_per_w)])

    return score_kernel


def kernel(entity_emb, relation_emb, pos_h, pos_r, pos_t):
    B = pos_h.shape[0]
    ent_wide = _tc_transpose(entity_emb.T)
    ent_rows = ent_wide.reshape(ent_wide.shape[0] * 2, _D)
    return _sc_score(B)(ent_rows, relation_emb, pos_h, pos_t, pos_r)


# speedup vs baseline: 1.0019x; 1.0019x over previous
import dataclasses
import functools

import jax
import jax.numpy as jnp
from jax import lax
from jax.experimental import pallas as pl
from jax.experimental.pallas import tpu as pltpu
from jax.experimental.pallas import tpu_sc as plsc

_NC = 2
_NS = 16
_NW = _NC * _NS
_D = 64
_L = 16
_CHUNK = 128
_TBL = 32768
_THB = _TBL // 2


def _tc_transpose_body(in_ref, out_ref):
    h = _TBL // 2
    a = in_ref[:, 0:h][...].T
    b = in_ref[:, h:_TBL][...].T
    out_ref[...] = jnp.concatenate([a, b], axis=1)


def _tc_transpose(ent_t):
    d, n = ent_t.shape
    n_blocks = (n + _TBL - 1) // _TBL
    return pl.pallas_call(
        _tc_transpose_body,
        grid=(n_blocks,),
        in_specs=[pl.BlockSpec((d, _TBL), lambda i: (0, i))],
        out_specs=pl.BlockSpec((_TBL // 2, 2 * d), lambda i: (i, 0)),
        out_shape=jax.ShapeDtypeStruct((n_blocks * (_TBL // 2), 2 * d),
                                       jnp.float32),
    )(ent_t)


def _sc_score(B):
    b_per_w = B // _NW
    n_chunks = b_per_w // _CHUNK
    mesh = plsc.VectorSubcoreMesh(core_axis_name="c", subcore_axis_name="s")

    cp = pltpu.CompilerParams(use_tc_tiling_on_sc=False)
    if "needs_layout_passes" in pltpu.CompilerParams.__dataclass_fields__:
        cp = dataclasses.replace(cp, needs_layout_passes=False)

    @functools.partial(
        pl.kernel,
        mesh=mesh,
        compiler_params=cp,
        out_type=jax.ShapeDtypeStruct((B,), jnp.float32),
        scratch_types=[
            pltpu.VMEM((b_per_w,), jnp.int32),
            pltpu.VMEM((b_per_w,), jnp.int32),
            pltpu.VMEM((b_per_w,), jnp.int32),
            pltpu.VMEM((b_per_w, _D), jnp.float32),
            pltpu.VMEM((b_per_w, _D), jnp.float32),
            pltpu.VMEM((b_per_w, _D), jnp.float32),
            pltpu.VMEM((b_per_w,), jnp.float32),
            pltpu.SemaphoreType.DMA,
        ],
    )
    def score_kernel(ent_hbm, rel_hbm, hidx_hbm, tidx_hbm, ridx_hbm, out_hbm,
                     hi_v, ti_v, ri_v,
                     hrow_v, trow_v, rrow_v, out_v, sem):
        wid = lax.axis_index("s") * _NC + lax.axis_index("c")
        base = wid * b_per_w
        src = pl.ds(base, b_per_w)
        pltpu.sync_copy(hidx_hbm.at[src], hi_v)
        pltpu.sync_copy(tidx_hbm.at[src], ti_v)
        pltpu.sync_copy(ridx_hbm.at[src], ri_v)
        def remap(i):
            b = lax.shift_right_logical(i, 15)
            l = i - b * _TBL
            half = jnp.where(l >= _THB, 1, 0).astype(jnp.int32)
            lm = l - half * _THB
            return lax.shift_left(b * _THB + lm, 1) + half

        for s in range(b_per_w // _L):
            sl = pl.ds(s * _L, _L)
            hi_v[sl] = remap(hi_v[sl])
            ti_v[sl] = remap(ti_v[sl])

        iota = lax.iota(jnp.int32, _L)

        copies = []
        for c in range(n_chunks):
            csl = pl.ds(c * _CHUNK, _CHUNK)
            copies.append(pltpu.async_copy(
                ent_hbm.at[hi_v.at[csl]], hrow_v.at[csl], sem))
            copies.append(pltpu.async_copy(
                ent_hbm.at[ti_v.at[csl]], trow_v.at[csl], sem))
            copies.append(pltpu.async_copy(
                rel_hbm.at[ri_v.at[csl]], rrow_v.at[csl], sem))
        for cpy in copies:
            cpy.wait()

        @pl.loop(0, b_per_w // _L)
        def _(g):
            lanes = iota + g * _L
            rsl = pl.ds(g * _L, _L)
            out_v[rsl] = jnp.zeros((_L,), jnp.float32)

            @pl.loop(0, _D, step=8)
            def _(d0):
                acc = jnp.zeros((_L,), jnp.float32)
                for dd in range(8):
                    dv = jnp.zeros((_L,), jnp.int32) + (d0 + dd)
                    hv = plsc.load_gather(hrow_v, [lanes, dv])
                    tv = plsc.load_gather(trow_v, [lanes, dv])
                    rv = plsc.load_gather(rrow_v, [lanes, dv])
                    acc = acc + jnp.abs(hv + rv - tv)
                plsc.addupdate(out_v.at[rsl], acc)

        pltpu.sync_copy(out_v, out_hbm.at[pl.ds(base, b_per_w)])

    return score_kernel


def kernel(entity_emb, relation_emb, pos_h, pos_r, pos_t):
    B = pos_h.shape[0]
    ent_wide = _tc_transpose(entity_emb.T)
    ent_rows = ent_wide.reshape(ent_wide.shape[0] * 2, _D)
    return _sc_score(B)(ent_rows, relation_emb, pos_h, pos_t, pos_r)


# speedup vs baseline: 1.0115x; 1.0095x over previous
import dataclasses
import functools

import jax
import jax.numpy as jnp
from jax import lax
from jax.experimental import pallas as pl
from jax.experimental.pallas import tpu as pltpu
from jax.experimental.pallas import tpu_sc as plsc

_NC = 2
_NS = 16
_NW = _NC * _NS
_D = 64
_L = 16
_CHUNK = 64
_TBL = 32768
_THB = _TBL // 2


def _tc_transpose_body(in_ref, out_ref):
    h = _TBL // 2
    a = in_ref[:, 0:h][...].T
    b = in_ref[:, h:_TBL][...].T
    out_ref[...] = jnp.concatenate([a, b], axis=1)


def _tc_transpose(ent_t):
    d, n = ent_t.shape
    n_blocks = (n + _TBL - 1) // _TBL
    return pl.pallas_call(
        _tc_transpose_body,
        grid=(n_blocks,),
        in_specs=[pl.BlockSpec((d, _TBL), lambda i: (0, i))],
        out_specs=pl.BlockSpec((_TBL // 2, 2 * d), lambda i: (i, 0)),
        out_shape=jax.ShapeDtypeStruct((n_blocks * (_TBL // 2), 2 * d),
                                       jnp.float32),
    )(ent_t)


def _sc_score(B):
    b_per_w = B // _NW
    n_chunks = b_per_w // _CHUNK
    mesh = plsc.VectorSubcoreMesh(core_axis_name="c", subcore_axis_name="s")

    cp = pltpu.CompilerParams(use_tc_tiling_on_sc=False)
    if "needs_layout_passes" in pltpu.CompilerParams.__dataclass_fields__:
        cp = dataclasses.replace(cp, needs_layout_passes=False)

    @functools.partial(
        pl.kernel,
        mesh=mesh,
        compiler_params=cp,
        out_type=jax.ShapeDtypeStruct((B,), jnp.float32),
        scratch_types=[
            pltpu.VMEM((b_per_w,), jnp.int32),
            pltpu.VMEM((b_per_w,), jnp.int32),
            pltpu.VMEM((b_per_w,), jnp.int32),
            pltpu.VMEM((b_per_w, _D), jnp.float32),
            pltpu.VMEM((b_per_w, _D), jnp.float32),
            pltpu.VMEM((b_per_w, _D), jnp.float32),
            pltpu.VMEM((b_per_w,), jnp.float32),
            pltpu.SemaphoreType.DMA,
        ],
    )
    def score_kernel(ent_hbm, rel_hbm, hidx_hbm, tidx_hbm, ridx_hbm, out_hbm,
                     hi_v, ti_v, ri_v,
                     hrow_v, trow_v, rrow_v, out_v, sem):
        wid = lax.axis_index("s") * _NC + lax.axis_index("c")
        base = wid * b_per_w
        src = pl.ds(base, b_per_w)
        icp1 = pltpu.async_copy(hidx_hbm.at[src], hi_v, sem)
        icp2 = pltpu.async_copy(tidx_hbm.at[src], ti_v, sem)
        icp3 = pltpu.async_copy(ridx_hbm.at[src], ri_v, sem)
        icp1.wait()
        icp2.wait()
        icp3.wait()
        def remap(i):
            b = lax.shift_right_logical(i, 15)
            l = i - b * _TBL
            half = jnp.where(l >= _THB, 1, 0).astype(jnp.int32)
            lm = l - half * _THB
            return lax.shift_left(b * _THB + lm, 1) + half

        for s in range(b_per_w // _L):
            sl = pl.ds(s * _L, _L)
            hi_v[sl] = remap(hi_v[sl])
            ti_v[sl] = remap(ti_v[sl])

        iota = lax.iota(jnp.int32, _L)

        copies = []
        for c in range(n_chunks):
            csl = pl.ds(c * _CHUNK, _CHUNK)
            copies.append(pltpu.async_copy(
                ent_hbm.at[hi_v.at[csl]], hrow_v.at[csl], sem))
            copies.append(pltpu.async_copy(
                ent_hbm.at[ti_v.at[csl]], trow_v.at[csl], sem))
            copies.append(pltpu.async_copy(
                rel_hbm.at[ri_v.at[csl]], rrow_v.at[csl], sem))
        for cpy in copies:
            cpy.wait()

        @pl.loop(0, b_per_w // _L)
        def _(g):
            lanes = iota + g * _L
            rsl = pl.ds(g * _L, _L)
            out_v[rsl] = jnp.zeros((_L,), jnp.float32)

            @pl.loop(0, _D, step=16)
            def _(d0):
                acc0 = jnp.zeros((_L,), jnp.float32)
                acc1 = jnp.zeros((_L,), jnp.float32)
                for dd in range(0, 16, 2):
                    dv0 = jnp.zeros((_L,), jnp.int32) + (d0 + dd)
                    dv1 = jnp.zeros((_L,), jnp.int32) + (d0 + dd + 1)
                    hv0 = plsc.load_gather(hrow_v, [lanes, dv0])
                    tv0 = plsc.load_gather(trow_v, [lanes, dv0])
                    rv0 = plsc.load_gather(rrow_v, [lanes, dv0])
                    hv1 = plsc.load_gather(hrow_v, [lanes, dv1])
                    tv1 = plsc.load_gather(trow_v, [lanes, dv1])
                    rv1 = plsc.load_gather(rrow_v, [lanes, dv1])
                    acc0 = acc0 + jnp.abs(hv0 + rv0 - tv0)
                    acc1 = acc1 + jnp.abs(hv1 + rv1 - tv1)
                plsc.addupdate(out_v.at[rsl], acc0 + acc1)

        pltpu.sync_copy(out_v, out_hbm.at[pl.ds(base, b_per_w)])

    return score_kernel


def kernel(entity_emb, relation_emb, pos_h, pos_r, pos_t):
    B = pos_h.shape[0]
    ent_wide = _tc_transpose(entity_emb.T)
    ent_rows = ent_wide.reshape(ent_wide.shape[0] * 2, _D)
    return _sc_score(B)(ent_rows, relation_emb, pos_h, pos_t, pos_r)


# speedup vs baseline: 1.0119x; 1.0004x over previous
import dataclasses
import functools

import jax
import jax.numpy as jnp
from jax import lax
from jax.experimental import pallas as pl
from jax.experimental.pallas import tpu as pltpu
from jax.experimental.pallas import tpu_sc as plsc

_NC = 2
_NS = 16
_NW = _NC * _NS
_D = 64
_L = 16
_CHUNK = 32
_TBL = 32768
_THB = _TBL // 2


def _tc_transpose_body(in_ref, out_ref):
    h = _TBL // 2
    a = in_ref[:, 0:h][...].T
    b = in_ref[:, h:_TBL][...].T
    out_ref[...] = jnp.concatenate([a, b], axis=1)


def _tc_transpose(ent_t):
    d, n = ent_t.shape
    n_blocks = (n + _TBL - 1) // _TBL
    return pl.pallas_call(
        _tc_transpose_body,
        grid=(n_blocks,),
        in_specs=[pl.BlockSpec((d, _TBL), lambda i: (0, i))],
        out_specs=pl.BlockSpec((_TBL // 2, 2 * d), lambda i: (i, 0)),
        out_shape=jax.ShapeDtypeStruct((n_blocks * (_TBL // 2), 2 * d),
                                       jnp.float32),
    )(ent_t)


def _sc_score(B):
    b_per_w = B // _NW
    n_chunks = b_per_w // _CHUNK
    mesh = plsc.VectorSubcoreMesh(core_axis_name="c", subcore_axis_name="s")

    cp = pltpu.CompilerParams(use_tc_tiling_on_sc=False)
    if "needs_layout_passes" in pltpu.CompilerParams.__dataclass_fields__:
        cp = dataclasses.replace(cp, needs_layout_passes=False)

    @functools.partial(
        pl.kernel,
        mesh=mesh,
        compiler_params=cp,
        out_type=jax.ShapeDtypeStruct((B,), jnp.float32),
        scratch_types=[
            pltpu.VMEM((b_per_w,), jnp.int32),
            pltpu.VMEM((b_per_w,), jnp.int32),
            pltpu.VMEM((b_per_w,), jnp.int32),
            pltpu.VMEM((b_per_w, _D), jnp.float32),
            pltpu.VMEM((b_per_w, _D), jnp.float32),
            pltpu.VMEM((b_per_w, _D), jnp.float32),
            pltpu.VMEM((b_per_w,), jnp.float32),
            pltpu.SemaphoreType.DMA,
        ],
    )
    def score_kernel(ent_hbm, rel_hbm, hidx_hbm, tidx_hbm, ridx_hbm, out_hbm,
                     hi_v, ti_v, ri_v,
                     hrow_v, trow_v, rrow_v, out_v, sem):
        wid = lax.axis_index("s") * _NC + lax.axis_index("c")
        base = wid * b_per_w
        src = pl.ds(base, b_per_w)
        icp1 = pltpu.async_copy(hidx_hbm.at[src], hi_v, sem)
        icp2 = pltpu.async_copy(tidx_hbm.at[src], ti_v, sem)
        icp3 = pltpu.async_copy(ridx_hbm.at[src], ri_v, sem)
        icp1.wait()
        icp2.wait()
        icp3.wait()
        def remap(i):
            b = lax.shift_right_logical(i, 15)
            l = i - b * _TBL
            half = jnp.where(l >= _THB, 1, 0).astype(jnp.int32)
            lm = l - half * _THB
            return lax.shift_left(b * _THB + lm, 1) + half

        for s in range(b_per_w // _L):
            sl = pl.ds(s * _L, _L)
            hi_v[sl] = remap(hi_v[sl])
            ti_v[sl] = remap(ti_v[sl])

        iota = lax.iota(jnp.int32, _L)

        copies = []
        for c in range(n_chunks):
            csl = pl.ds(c * _CHUNK, _CHUNK)
            copies.append(pltpu.async_copy(
                ent_hbm.at[hi_v.at[csl]], hrow_v.at[csl], sem))
            copies.append(pltpu.async_copy(
                ent_hbm.at[ti_v.at[csl]], trow_v.at[csl], sem))
            copies.append(pltpu.async_copy(
                rel_hbm.at[ri_v.at[csl]], rrow_v.at[csl], sem))
        for cpy in copies:
            cpy.wait()

        @pl.loop(0, b_per_w // _L)
        def _(g):
            lanes = iota + g * _L
            rsl = pl.ds(g * _L, _L)
            out_v[rsl] = jnp.zeros((_L,), jnp.float32)

            @pl.loop(0, _D, step=16)
            def _(d0):
                acc0 = jnp.zeros((_L,), jnp.float32)
                acc1 = jnp.zeros((_L,), jnp.float32)
                for dd in range(0, 16, 2):
                    dv0 = jnp.zeros((_L,), jnp.int32) + (d0 + dd)
                    dv1 = jnp.zeros((_L,), jnp.int32) + (d0 + dd + 1)
                    hv0 = plsc.load_gather(hrow_v, [lanes, dv0])
                    tv0 = plsc.load_gather(trow_v, [lanes, dv0])
                    rv0 = plsc.load_gather(rrow_v, [lanes, dv0])
                    hv1 = plsc.load_gather(hrow_v, [lanes, dv1])
                    tv1 = plsc.load_gather(trow_v, [lanes, dv1])
                    rv1 = plsc.load_gather(rrow_v, [lanes, dv1])
                    acc0 = acc0 + jnp.abs(hv0 + rv0 - tv0)
                    acc1 = acc1 + jnp.abs(hv1 + rv1 - tv1)
                plsc.addupdate(out_v.at[rsl], acc0 + acc1)

        pltpu.sync_copy(out_v, out_hbm.at[pl.ds(base, b_per_w)])

    return score_kernel


def kernel(entity_emb, relation_emb, pos_h, pos_r, pos_t):
    B = pos_h.shape[0]
    ent_wide = _tc_transpose(entity_emb.T)
    ent_rows = ent_wide.reshape(ent_wide.shape[0] * 2, _D)
    return _sc_score(B)(ent_rows, relation_emb, pos_h, pos_t, pos_r)
